# baseline (speedup 1.0000x reference)
import functools

import jax
import jax.numpy as jnp
from jax import lax
from jax.experimental import pallas as pl
from jax.experimental.pallas import tpu as pltpu
from jax.experimental.pallas import tpu_sc as plsc

N = 10000
E = 160000
F = 128
H = 256
OUT = 128
L = 3
NS = 7
NT = 3
NREL = NS + NT
NMAT = NREL + 2
NBASIS = 9

NB = 10
N_BLK = N // NB

NTILE = 32
NCHUNK = 160
CK = (2 * N) // NCHUNK
CPT = NCHUNK // NTILE
KE = 128
EPAD = 2 * KE
NEST = 176


def _proj_body(x_ref, wp_ref, bp_ref, o_ref):
    o_ref[...] = (
        jnp.dot(x_ref[...], wp_ref[...], preferred_element_type=jnp.float32)
        + bp_ref[...]
    )


def _proj(x, Wp, bp_row):
    return pl.pallas_call(
        _proj_body,
        grid=(NB,),
        in_specs=[
            pl.BlockSpec((N_BLK, F), lambda nb_: (nb_, 0)),
            pl.BlockSpec((F, H), lambda nb_: (0, 0)),
            pl.BlockSpec((1, H), lambda nb_: (0, 0)),
        ],
        out_specs=pl.BlockSpec((N_BLK, H), lambda nb_: (nb_, 0)),
        out_shape=jax.ShapeDtypeStruct((N, H), jnp.float32),
    )(x, Wp, bp_row)


def _wstack_body(c_ref, bas_ref, o_ref):
    acc = c_ref[0, 0, 0] * bas_ref[0]
    for b in range(1, NBASIS):
        acc = acc + c_ref[0, 0, b] * bas_ref[b]
    o_ref[0] = acc


def _wstack(Ci, BASi):
    return pl.pallas_call(
        _wstack_body,
        grid=(NMAT,),
        in_specs=[
            pl.BlockSpec((1, 1, NBASIS), lambda r: (r, 0, 0)),
            pl.BlockSpec((NBASIS, H, H), lambda r: (0, 0, 0)),
        ],
        out_specs=pl.BlockSpec((1, H, H), lambda r: (r, 0, 0)),
        out_shape=jax.ShapeDtypeStruct((NMAT, H, H), jnp.float32),
    )(Ci.reshape(NMAT, 1, NBASIS), BASi)


def _tables_body(h_ref, w_ref, o_ref):
    o_ref[...] = jnp.dot(h_ref[...], w_ref[0], preferred_element_type=jnp.float32)


def _tables(h, Wstack):
    return pl.pallas_call(
        _tables_body,
        grid=(NB, NMAT),
        in_specs=[
            pl.BlockSpec((N_BLK, H), lambda nb_, r: (nb_, 0)),
            pl.BlockSpec((1, H, H), lambda nb_, r: (r, 0, 0)),
        ],
        out_specs=pl.BlockSpec((N_BLK, H), lambda nb_, r: (r * NB + nb_, 0)),
        out_shape=jax.ShapeDtypeStruct((NMAT * N, H), jnp.float32),
    )(h, Wstack)


def _fuse_body(aggs_ref, aggt_ref, roots_ref, roott_ref, fw_ref, fb_ref,
               sb_ref, tb_ref, ng_ref, nbb_ref, hprev_ref, o_ref, *, first):
    fw_top = fw_ref[:H, :]
    fw_bot = fw_ref[H:, :]
    hs = aggs_ref[...] + roots_ref[...]
    ht = aggt_ref[...] + roott_ref[...]
    z = (
        jnp.dot(hs, fw_top, preferred_element_type=jnp.float32)
        + jnp.dot(ht, fw_bot, preferred_element_type=jnp.float32)
        + jnp.dot(sb_ref[...], fw_top, preferred_element_type=jnp.float32)
        + jnp.dot(tb_ref[...], fw_bot, preferred_element_type=jnp.float32)
        + fb_ref[...]
    )
    mu = jnp.mean(z, axis=-1, keepdims=True)
    d = z - mu
    var = jnp.mean(d * d, axis=-1, keepdims=True)
    zn = d / jnp.sqrt(var + 1e-5) * ng_ref[...] + nbb_ref[...]
    hn = jnp.maximum(zn, 0.0)
    if first:
        o_ref[...] = hn
    else:
        o_ref[...] = hn + hprev_ref[...]


def _fuse(agg, table, fWi, fbi, sbi, tbi, ngi, nbi, h_prev, first):
    return pl.pallas_call(
        functools.partial(_fuse_body, first=first),
        grid=(NB,),
        in_specs=[
            pl.BlockSpec((N_BLK, H), lambda nb_: (nb_, 0)),
            pl.BlockSpec((N_BLK, H), lambda nb_: (NB + nb_, 0)),
            pl.BlockSpec((N_BLK, H), lambda nb_: (NREL * NB + nb_, 0)),
            pl.BlockSpec((N_BLK, H), lambda nb_: ((NREL + 1) * NB + nb_, 0)),
            pl.BlockSpec((2 * H, H), lambda nb_: (0, 0)),
            pl.BlockSpec((1, H), lambda nb_: (0, 0)),
            pl.BlockSpec((1, H), lambda nb_: (0, 0)),
            pl.BlockSpec((1, H), lambda nb_: (0, 0)),
            pl.BlockSpec((1, H), lambda nb_: (0, 0)),
            pl.BlockSpec((1, H), lambda nb_: (0, 0)),
            pl.BlockSpec((N_BLK, H), lambda nb_: (nb_, 0)),
        ],
        out_specs=pl.BlockSpec((N_BLK, H), lambda nb_: (nb_, 0)),
        out_shape=jax.ShapeDtypeStruct((N, H), jnp.float32),
    )(agg, agg, table, table, fWi,
      fbi.reshape(1, H), sbi.reshape(1, H), tbi.reshape(1, H),
      ngi.reshape(1, H), nbi.reshape(1, H), h_prev)


def _head_body(h_ref, w1_ref, b1_ref, w2_ref, b2_ref, o_ref):
    z = jnp.maximum(
        jnp.dot(h_ref[...], w1_ref[...], preferred_element_type=jnp.float32)
        + b1_ref[...],
        0.0,
    )
    o_ref[...] = (
        jnp.dot(z, w2_ref[...], preferred_element_type=jnp.float32) + b2_ref[...]
    )


def _head(h, hW1, hb1_row, hW2, hb2_row):
    return pl.pallas_call(
        _head_body,
        grid=(NB,),
        in_specs=[
            pl.BlockSpec((N_BLK, H), lambda nb_: (nb_, 0)),
            pl.BlockSpec((H, H), lambda nb_: (0, 0)),
            pl.BlockSpec((1, H), lambda nb_: (0, 0)),
            pl.BlockSpec((H, OUT), lambda nb_: (0, 0)),
            pl.BlockSpec((1, OUT), lambda nb_: (0, 0)),
        ],
        out_specs=pl.BlockSpec((N_BLK, OUT), lambda nb_: (nb_, 0)),
        out_shape=jax.ShapeDtypeStruct((N, OUT), jnp.float32),
    )(h, hW1, hb1_row, hW2, hb2_row)


def _sc_agg_body(table, gidx, keys, invs, estart, out,
                 est_v, idxA, rowsA, idxB, rowsB, key_v, inv_v, acc_v,
                 semA, semB):
    wid = lax.axis_index("s") * 2 + lax.axis_index("c")
    pltpu.sync_copy(estart, est_v)

    def cbody(i, _):
        c = wid * CPT + i
        kbase = c * CK
        ev = est_v[pl.ds(c, 16)]
        e0 = ev[0]
        e1 = ev[1]

        def zbody(z, _):
            acc_v[pl.ds(z * 16, 16)] = jnp.zeros((16,), jnp.float32)
            return 0
        lax.fori_loop(0, CK * H // 16, zbody, 0)

        e0a = (e0 // 8) * 8
        nsub = est_v[pl.ds(NCHUNK, 16)][1]

        def issue(s, idx_b, rows_b, sem):
            base = jnp.minimum(e0a + s * KE, E)
            pltpu.sync_copy(gidx.at[pl.ds(base, KE)], idx_b)
            pltpu.async_copy(table.at[idx_b], rows_b, sem)

        def process(s, rows_b):
            base = jnp.minimum(e0a + s * KE, E)
            pltpu.sync_copy(keys.at[pl.ds(base, KE)], key_v)
            pltpu.sync_copy(invs.at[pl.ds(base, KE)], inv_v)

            def gbody(g, _):
                k16 = key_v[pl.ds(g * 16, 16)]
                i16 = inv_v[pl.ds(g * 16, 16)]
                eb = base + g * 16
                for j in range(16):
                    e = eb + j
                    valid = (e >= e0) & (e < e1)
                    sc_ = jnp.where(valid, i16[j], 0.0)
                    off = jnp.clip(k16[j] - kbase, 0, CK - 1) * H
                    sv = jnp.full((16,), sc_, jnp.float32)
                    for t in range(H // 16):
                        row = rows_b[eb - base + j, pl.ds(t * 16, 16)]
                        plsc.addupdate(
                            acc_v.at[pl.ds(off + t * 16, 16)], row * sv)
                return 0
            lax.fori_loop(0, KE // 16, gbody, 0)

        @pl.when(nsub > 0)
        def _():
            issue(0, idxA, rowsA, semA)

        def pair_body(p, _):
            sA = 2 * p
            sB = sA + 1

            @pl.when(sB < nsub)
            def _():
                issue(sB, idxB, rowsB, semB)

            pltpu.make_async_copy(table.at[idxA], rowsA, semA).wait()
            process(sA, rowsA)

            @pl.when(sA + 2 < nsub)
            def _():
                issue(sA + 2, idxA, rowsA, semA)

            @pl.when(sB < nsub)
            def _():
                pltpu.make_async_copy(table.at[idxB], rowsB, semB).wait()
                process(sB, rowsB)
            return 0
        lax.fori_loop(0, (nsub + 1) // 2, pair_body, 0)

        pltpu.sync_copy(acc_v, out.at[pl.ds(kbase * H, CK * H)])
        return 0

    lax.fori_loop(0, CPT, cbody, 0)


def _sc_agg(table, gidx, keys, invs, estart):
    mesh = plsc.VectorSubcoreMesh(core_axis_name="c", subcore_axis_name="s")
    return pl.kernel(
        _sc_agg_body,
        out_type=jax.ShapeDtypeStruct((2 * N * H,), jnp.float32),
        mesh=mesh,
        scratch_types=[
            pltpu.VMEM((NEST,), jnp.int32),
            pltpu.VMEM((KE,), jnp.int32),
            pltpu.VMEM((KE, H), jnp.float32),
            pltpu.VMEM((KE,), jnp.int32),
            pltpu.VMEM((KE, H), jnp.float32),
            pltpu.VMEM((KE,), jnp.int32),
            pltpu.VMEM((KE,), jnp.float32),
            pltpu.VMEM((CK * H,), jnp.float32),
            pltpu.SemaphoreType.DMA,
            pltpu.SemaphoreType.DMA,
        ],
    )(table, gidx, keys, invs, estart)


def kernel(x, edge_index, edge_type, Wp, bp, sbasis, scomp, sroot, sbias,
           tbasis, tcomp, troot, tbias, fW, fb, ng, nb, hW1, hb1, hW2, hb2):
    f32 = jnp.float32
    src = edge_index[0].astype(jnp.int32)
    dst = edge_index[1].astype(jnp.int32)
    et = edge_type.astype(jnp.int32)

    tem = (et >= NS).astype(jnp.int32)
    keyv = tem * N + dst
    order = jnp.argsort(keyv)
    gidx = (et * N + src)[order]
    keys = keyv[order]
    cnt = jnp.zeros((NREL * N,), f32).at[et * N + dst].add(1.0)
    inv_e = 1.0 / jnp.maximum(cnt[et * N + dst], 1.0)
    invs = inv_e[order]
    bound = jnp.arange(NCHUNK + 1, dtype=jnp.int32) * CK
    estart = jnp.searchsorted(keys, bound).astype(jnp.int32)
    e0a_all = (estart[:NCHUNK] // 8) * 8
    nsub_max = jnp.max(
        (estart[1:NCHUNK + 1] - e0a_all + KE - 1) // KE).astype(jnp.int32)
    estart = jnp.concatenate(
        [estart, nsub_max[None],
         jnp.full((NEST - NCHUNK - 2,), E, jnp.int32)])
    gidx = jnp.concatenate([gidx, jnp.zeros((EPAD,), jnp.int32)])
    keys = jnp.concatenate([keys, jnp.zeros((EPAD,), jnp.int32)])
    invs = jnp.concatenate([invs, jnp.zeros((EPAD,), f32)])

    C = jnp.zeros((L, NMAT, NBASIS), f32)
    C = C.at[:, :NS, :4].set(scomp)
    C = C.at[:, NS:NREL, 4:7].set(tcomp)
    C = C.at[:, NREL, 7].set(1.0)
    C = C.at[:, NREL + 1, 8].set(1.0)
    BAS = jnp.concatenate(
        [sbasis, tbasis, sroot[:, None], troot[:, None]], axis=1)

    h = _proj(x, Wp, bp.reshape(1, H))
    for i in range(L):
        Wstack = _wstack(C[i], BAS[i])
        table = _tables(h, Wstack)
        agg = _sc_agg(table, gidx, keys, invs, estart).reshape(2 * N, H)
        h = _fuse(agg, table, fW[i], fb[i], sbias[i], tbias[i],
                  ng[i], nb[i], h, first=(i == 0))
    return _head(h, hW1, hb1.reshape(1, H), hW2, hb2.reshape(1, OUT))


# speedup vs baseline: 1.0212x; 1.0212x over previous
import functools

import jax
import jax.numpy as jnp
from jax import lax
from jax.experimental import pallas as pl
from jax.experimental.pallas import tpu as pltpu
from jax.experimental.pallas import tpu_sc as plsc

N = 10000
E = 160000
F = 128
H = 256
OUT = 128
L = 3
NS = 7
NT = 3
NREL = NS + NT
NMAT = NREL + 2
NBASIS = 9

NB = 10
N_BLK = N // NB

NTILE = 32
NCHUNK = 160
CK = (2 * N) // NCHUNK
CPT = NCHUNK // NTILE
KE = 128
EPAD = 2 * KE
NEST = 176


def _proj_body(x_ref, wp_ref, bp_ref, o_ref):
    o_ref[...] = (
        jnp.dot(x_ref[...], wp_ref[...], preferred_element_type=jnp.float32)
        + bp_ref[...]
    )


def _proj(x, Wp, bp_row):
    return pl.pallas_call(
        _proj_body,
        grid=(NB,),
        in_specs=[
            pl.BlockSpec((N_BLK, F), lambda nb_: (nb_, 0)),
            pl.BlockSpec((F, H), lambda nb_: (0, 0)),
            pl.BlockSpec((1, H), lambda nb_: (0, 0)),
        ],
        out_specs=pl.BlockSpec((N_BLK, H), lambda nb_: (nb_, 0)),
        out_shape=jax.ShapeDtypeStruct((N, H), jnp.float32),
    )(x, Wp, bp_row)


def _wstack_body(c_ref, bas_ref, o_ref):
    acc = c_ref[0, 0, 0] * bas_ref[0]
    for b in range(1, NBASIS):
        acc = acc + c_ref[0, 0, b] * bas_ref[b]
    o_ref[0] = acc


def _wstack(Ci, BASi):
    return pl.pallas_call(
        _wstack_body,
        grid=(NMAT,),
        in_specs=[
            pl.BlockSpec((1, 1, NBASIS), lambda r: (r, 0, 0)),
            pl.BlockSpec((NBASIS, H, H), lambda r: (0, 0, 0)),
        ],
        out_specs=pl.BlockSpec((1, H, H), lambda r: (r, 0, 0)),
        out_shape=jax.ShapeDtypeStruct((NMAT, H, H), jnp.float32),
    )(Ci.reshape(NMAT, 1, NBASIS), BASi)


def _tables_body(h_ref, w_ref, o_ref):
    o_ref[...] = jnp.dot(h_ref[...], w_ref[0], preferred_element_type=jnp.float32)


def _tables(h, Wstack):
    return pl.pallas_call(
        _tables_body,
        grid=(NB, NMAT),
        in_specs=[
            pl.BlockSpec((N_BLK, H), lambda nb_, r: (nb_, 0)),
            pl.BlockSpec((1, H, H), lambda nb_, r: (r, 0, 0)),
        ],
        out_specs=pl.BlockSpec((N_BLK, H), lambda nb_, r: (r * NB + nb_, 0)),
        out_shape=jax.ShapeDtypeStruct((NMAT * N, H), jnp.float32),
    )(h, Wstack)


def _fuse_body(aggs_ref, aggt_ref, roots_ref, roott_ref, fw_ref, fb_ref,
               sb_ref, tb_ref, ng_ref, nbb_ref, hprev_ref, o_ref, *, first):
    fw_top = fw_ref[:H, :]
    fw_bot = fw_ref[H:, :]
    hs = aggs_ref[...] + roots_ref[...]
    ht = aggt_ref[...] + roott_ref[...]
    z = (
        jnp.dot(hs, fw_top, preferred_element_type=jnp.float32)
        + jnp.dot(ht, fw_bot, preferred_element_type=jnp.float32)
        + jnp.dot(sb_ref[...], fw_top, preferred_element_type=jnp.float32)
        + jnp.dot(tb_ref[...], fw_bot, preferred_element_type=jnp.float32)
        + fb_ref[...]
    )
    mu = jnp.mean(z, axis=-1, keepdims=True)
    d = z - mu
    var = jnp.mean(d * d, axis=-1, keepdims=True)
    zn = d / jnp.sqrt(var + 1e-5) * ng_ref[...] + nbb_ref[...]
    hn = jnp.maximum(zn, 0.0)
    if first:
        o_ref[...] = hn
    else:
        o_ref[...] = hn + hprev_ref[...]


def _fuse(agg, table, fWi, fbi, sbi, tbi, ngi, nbi, h_prev, first):
    return pl.pallas_call(
        functools.partial(_fuse_body, first=first),
        grid=(NB,),
        in_specs=[
            pl.BlockSpec((N_BLK, H), lambda nb_: (nb_, 0)),
            pl.BlockSpec((N_BLK, H), lambda nb_: (NB + nb_, 0)),
            pl.BlockSpec((N_BLK, H), lambda nb_: (NREL * NB + nb_, 0)),
            pl.BlockSpec((N_BLK, H), lambda nb_: ((NREL + 1) * NB + nb_, 0)),
            pl.BlockSpec((2 * H, H), lambda nb_: (0, 0)),
            pl.BlockSpec((1, H), lambda nb_: (0, 0)),
            pl.BlockSpec((1, H), lambda nb_: (0, 0)),
            pl.BlockSpec((1, H), lambda nb_: (0, 0)),
            pl.BlockSpec((1, H), lambda nb_: (0, 0)),
            pl.BlockSpec((1, H), lambda nb_: (0, 0)),
            pl.BlockSpec((N_BLK, H), lambda nb_: (nb_, 0)),
        ],
        out_specs=pl.BlockSpec((N_BLK, H), lambda nb_: (nb_, 0)),
        out_shape=jax.ShapeDtypeStruct((N, H), jnp.float32),
    )(agg, agg, table, table, fWi,
      fbi.reshape(1, H), sbi.reshape(1, H), tbi.reshape(1, H),
      ngi.reshape(1, H), nbi.reshape(1, H), h_prev)


def _head_body(h_ref, w1_ref, b1_ref, w2_ref, b2_ref, o_ref):
    z = jnp.maximum(
        jnp.dot(h_ref[...], w1_ref[...], preferred_element_type=jnp.float32)
        + b1_ref[...],
        0.0,
    )
    o_ref[...] = (
        jnp.dot(z, w2_ref[...], preferred_element_type=jnp.float32) + b2_ref[...]
    )


def _head(h, hW1, hb1_row, hW2, hb2_row):
    return pl.pallas_call(
        _head_body,
        grid=(NB,),
        in_specs=[
            pl.BlockSpec((N_BLK, H), lambda nb_: (nb_, 0)),
            pl.BlockSpec((H, H), lambda nb_: (0, 0)),
            pl.BlockSpec((1, H), lambda nb_: (0, 0)),
            pl.BlockSpec((H, OUT), lambda nb_: (0, 0)),
            pl.BlockSpec((1, OUT), lambda nb_: (0, 0)),
        ],
        out_specs=pl.BlockSpec((N_BLK, OUT), lambda nb_: (nb_, 0)),
        out_shape=jax.ShapeDtypeStruct((N, OUT), jnp.float32),
    )(h, hW1, hb1_row, hW2, hb2_row)


def _sc_agg_body(table, gidx, keys, invs, estart, out,
                 est_v, idxA, rowsA, idxB, rowsB, key_v, inv_v, acc_v,
                 semA, semB):
    wid = lax.axis_index("s") * 2 + lax.axis_index("c")
    pltpu.sync_copy(estart, est_v)

    def cbody(i, _):
        c = wid * CPT + i
        kbase = c * CK
        ev = est_v[pl.ds(c, 16)]
        e0 = ev[0]
        e1 = ev[1]

        def zbody(z, _):
            acc_v[pl.ds(z * 16, 16)] = jnp.zeros((16,), jnp.float32)
            return 0
        lax.fori_loop(0, CK * H // 16, zbody, 0)

        e0a = (e0 // 8) * 8
        nsub = (e1 - e0a + KE - 1) // KE

        def issue(s, idx_b, rows_b, sem):
            base = jnp.minimum(e0a + s * KE, E)
            pltpu.sync_copy(gidx.at[pl.ds(base, KE)], idx_b)
            pltpu.async_copy(table.at[idx_b], rows_b, sem)

        def process(s, rows_b):
            base = jnp.minimum(e0a + s * KE, E)
            pltpu.sync_copy(keys.at[pl.ds(base, KE)], key_v)
            pltpu.sync_copy(invs.at[pl.ds(base, KE)], inv_v)

            def gbody(g, _):
                k16 = key_v[pl.ds(g * 16, 16)]
                i16 = inv_v[pl.ds(g * 16, 16)]
                eb = base + g * 16
                ev16 = lax.broadcasted_iota(jnp.int32, (16,), 0) + eb
                valid = (ev16 >= e0) & (ev16 < e1)
                sc16 = jnp.where(valid, i16, jnp.zeros((16,), jnp.float32))
                off16 = jnp.clip(k16 - kbase, 0, CK - 1) * H
                for j in range(16):
                    off = off16[j]
                    sv = jnp.full((16,), sc16[j], jnp.float32)
                    for t in range(H // 16):
                        row = rows_b[eb - base + j, pl.ds(t * 16, 16)]
                        plsc.addupdate(
                            acc_v.at[pl.ds(off + t * 16, 16)], row * sv)
                return 0
            lax.fori_loop(0, KE // 16, gbody, 0)

        @pl.when(nsub > 0)
        def _():
            issue(0, idxA, rowsA, semA)

        def pair_body(p, _):
            sA = 2 * p
            sB = sA + 1

            @pl.when(sB < nsub)
            def _():
                issue(sB, idxB, rowsB, semB)

            pltpu.make_async_copy(table.at[idxA], rowsA, semA).wait()
            process(sA, rowsA)

            @pl.when(sA + 2 < nsub)
            def _():
                issue(sA + 2, idxA, rowsA, semA)

            @pl.when(sB < nsub)
            def _():
                pltpu.make_async_copy(table.at[idxB], rowsB, semB).wait()
                process(sB, rowsB)
            return 0
        lax.fori_loop(0, (nsub + 1) // 2, pair_body, 0)

        pltpu.sync_copy(acc_v, out.at[pl.ds(kbase * H, CK * H)])
        return 0

    lax.fori_loop(0, CPT, cbody, 0)


def _sc_agg(table, gidx, keys, invs, estart):
    mesh = plsc.VectorSubcoreMesh(core_axis_name="c", subcore_axis_name="s")
    return pl.kernel(
        _sc_agg_body,
        out_type=jax.ShapeDtypeStruct((2 * N * H,), jnp.float32),
        mesh=mesh,
        scratch_types=[
            pltpu.VMEM((NEST,), jnp.int32),
            pltpu.VMEM((KE,), jnp.int32),
            pltpu.VMEM((KE, H), jnp.float32),
            pltpu.VMEM((KE,), jnp.int32),
            pltpu.VMEM((KE, H), jnp.float32),
            pltpu.VMEM((KE,), jnp.int32),
            pltpu.VMEM((KE,), jnp.float32),
            pltpu.VMEM((CK * H,), jnp.float32),
            pltpu.SemaphoreType.DMA,
            pltpu.SemaphoreType.DMA,
        ],
    )(table, gidx, keys, invs, estart)


def kernel(x, edge_index, edge_type, Wp, bp, sbasis, scomp, sroot, sbias,
           tbasis, tcomp, troot, tbias, fW, fb, ng, nb, hW1, hb1, hW2, hb2):
    f32 = jnp.float32
    src = edge_index[0].astype(jnp.int32)
    dst = edge_index[1].astype(jnp.int32)
    et = edge_type.astype(jnp.int32)

    tem = (et >= NS).astype(jnp.int32)
    keyv = tem * N + dst
    order = jnp.argsort(keyv)
    gidx = (et * N + src)[order]
    keys = keyv[order]
    cnt = jnp.zeros((NREL * N,), f32).at[et * N + dst].add(1.0)
    inv_e = 1.0 / jnp.maximum(cnt[et * N + dst], 1.0)
    invs = inv_e[order]
    bound = jnp.arange(NCHUNK + 1, dtype=jnp.int32) * CK
    estart = jnp.searchsorted(keys, bound).astype(jnp.int32)
    e0a_all = (estart[:NCHUNK] // 8) * 8
    nsub_max = jnp.max(
        (estart[1:NCHUNK + 1] - e0a_all + KE - 1) // KE).astype(jnp.int32)
    estart = jnp.concatenate(
        [estart, nsub_max[None],
         jnp.full((NEST - NCHUNK - 2,), E, jnp.int32)])
    gidx = jnp.concatenate([gidx, jnp.zeros((EPAD,), jnp.int32)])
    keys = jnp.concatenate([keys, jnp.zeros((EPAD,), jnp.int32)])
    invs = jnp.concatenate([invs, jnp.zeros((EPAD,), f32)])

    C = jnp.zeros((L, NMAT, NBASIS), f32)
    C = C.at[:, :NS, :4].set(scomp)
    C = C.at[:, NS:NREL, 4:7].set(tcomp)
    C = C.at[:, NREL, 7].set(1.0)
    C = C.at[:, NREL + 1, 8].set(1.0)
    BAS = jnp.concatenate(
        [sbasis, tbasis, sroot[:, None], troot[:, None]], axis=1)

    h = _proj(x, Wp, bp.reshape(1, H))
    for i in range(L):
        Wstack = _wstack(C[i], BAS[i])
        table = _tables(h, Wstack)
        agg = _sc_agg(table, gidx, keys, invs, estart).reshape(2 * N, H)
        h = _fuse(agg, table, fW[i], fb[i], sbias[i], tbias[i],
                  ng[i], nb[i], h, first=(i == 0))
    return _head(h, hW1, hb1.reshape(1, H), hW2, hb2.reshape(1, OUT))
